# Initial kernel scaffold; baseline (speedup 1.0000x reference)
import jax
import jax.numpy as jnp
from jax.experimental import pallas as pl


def kernel(X, W_head, b_head, W_tail, b_tail, Wl0, bl0, Wr0, Wl1, bl1, Wr1, W_sum, b_sum, W_bi, b_bi, W_gu, b_gu, W_gv, b_gv, W_gw, b_gw, gamma, beta):
    raise NotImplementedError("write your pallas kernel here")


# speedup vs baseline: 3.1820x; 3.1820x over previous
import functools

import jax
import jax.numpy as jnp
from jax.experimental import pallas as pl
from jax.experimental.pallas import tpu as pltpu

N = 10000
D = 256
K = 16
NP = 10112
RB = 200
NBLK = N // RB
NEG = -1.0e30


def _mm_kernel(x_ref, wh_ref, bh_ref, wt_ref, bt_ref, eh_ref, et_ref):
    x = x_ref[...]
    eh_ref[...] = jnp.dot(x, wh_ref[...], preferred_element_type=jnp.float32) + bh_ref[...]
    et_ref[...] = jnp.dot(x, wt_ref[...], preferred_element_type=jnp.float32) + bt_ref[...]


def _heads(X, W_head, b_head, W_tail, b_tail):
    return pl.pallas_call(
        _mm_kernel,
        grid=(NBLK,),
        in_specs=[
            pl.BlockSpec((RB, D), lambda i: (i, 0)),
            pl.BlockSpec((D, D), lambda i: (0, 0)),
            pl.BlockSpec((1, D), lambda i: (0, 0)),
            pl.BlockSpec((D, D), lambda i: (0, 0)),
            pl.BlockSpec((1, D), lambda i: (0, 0)),
        ],
        out_specs=[
            pl.BlockSpec((RB, D), lambda i: (i, 0)),
            pl.BlockSpec((RB, D), lambda i: (i, 0)),
        ],
        out_shape=[
            jax.ShapeDtypeStruct((N, D), jnp.float32),
            jax.ShapeDtypeStruct((N, D), jnp.float32),
        ],
    )(X, W_head, b_head.reshape(1, D), W_tail, b_tail.reshape(1, D))


def _topk_kernel(eh_ref, etp_ref, w_ref, idx_ref):
    scale = D ** (-0.5)
    logits = jax.lax.dot_general(
        eh_ref[...], etp_ref[...], (((1,), (1,)), ((), ())),
        preferred_element_type=jnp.float32) * scale
    col = jax.lax.broadcasted_iota(jnp.int32, (RB, NP), 1)
    logits = jnp.where(col < N, logits, NEG)
    vals, idxs = [], []
    l = logits
    for _ in range(K):
        m = jnp.max(l, axis=1, keepdims=True)
        am = jnp.min(jnp.where(l == m, col, jnp.int32(NP)), axis=1, keepdims=True)
        vals.append(m)
        idxs.append(am)
        l = jnp.where(col == am, NEG, l)
    v = jnp.concatenate(vals, axis=1)
    ix = jnp.concatenate(idxs, axis=1)
    e = jnp.exp(v - v[:, 0:1])
    w_ref[...] = e / jnp.sum(e, axis=1, keepdims=True)
    idx_ref[...] = ix


def _topk(e_h, e_tp):
    return pl.pallas_call(
        _topk_kernel,
        grid=(NBLK,),
        in_specs=[
            pl.BlockSpec((RB, D), lambda i: (i, 0)),
            pl.BlockSpec((NP, D), lambda i: (0, 0)),
        ],
        out_specs=[
            pl.BlockSpec((RB, K), lambda i: (i, 0)),
            pl.BlockSpec((RB, K), lambda i: (i, 0)),
        ],
        out_shape=[
            jax.ShapeDtypeStruct((N, K), jnp.float32),
            jax.ShapeDtypeStruct((N, K), jnp.int32),
        ],
    )(e_h, e_tp)


def _scatter_kernel(idxt_ref, x_ref, sum_ref, cnt_ref):
    i = pl.program_id(0)

    @pl.when(i == 0)
    def _():
        sum_ref[...] = jnp.zeros_like(sum_ref)
        cnt_ref[...] = jnp.zeros_like(cnt_ref)

    j = jax.lax.broadcasted_iota(jnp.int32, (NP, RB), 0)
    oh = jnp.zeros((NP, RB), jnp.float32)
    for k in range(K):
        oh = oh + jnp.where(j == idxt_ref[0, k, :][None, :], 1.0, 0.0)
    sum_ref[...] += jax.lax.dot_general(
        oh, x_ref[...], (((1,), (0,)), ((), ())), preferred_element_type=jnp.float32)
    cnt_ref[...] += jnp.sum(oh, axis=1, keepdims=True)


def _scatter(idxt3, x):
    return pl.pallas_call(
        _scatter_kernel,
        grid=(NBLK,),
        in_specs=[
            pl.BlockSpec((1, K, RB), lambda i: (i, 0, 0)),
            pl.BlockSpec((RB, D), lambda i: (i, 0)),
        ],
        out_specs=[
            pl.BlockSpec((NP, D), lambda i: (0, 0)),
            pl.BlockSpec((NP, 1), lambda i: (0, 0)),
        ],
        out_shape=[
            jax.ShapeDtypeStruct((NP, D), jnp.float32),
            jax.ShapeDtypeStruct((NP, 1), jnp.float32),
        ],
        compiler_params=pltpu.CompilerParams(
            dimension_semantics=("arbitrary",)),
    )(idxt3, x)


def _sage_dense_kernel(aggr_ref, cnt_ref, x_ref, wl_ref, bl_ref, wr_ref, o_ref):
    aggr = aggr_ref[...] / jnp.maximum(cnt_ref[...], 1.0)
    y = (jnp.dot(aggr, wl_ref[...], preferred_element_type=jnp.float32)
         + bl_ref[...]
         + jnp.dot(x_ref[...], wr_ref[...], preferred_element_type=jnp.float32))
    o_ref[...] = jnp.where(y > 0, y, 0.01 * y)


def _sage_dense(aggr, cnt, x, Wl, bl, Wr):
    return pl.pallas_call(
        _sage_dense_kernel,
        grid=(NBLK,),
        in_specs=[
            pl.BlockSpec((RB, D), lambda i: (i, 0)),
            pl.BlockSpec((RB, 1), lambda i: (i, 0)),
            pl.BlockSpec((RB, D), lambda i: (i, 0)),
            pl.BlockSpec((D, D), lambda i: (0, 0)),
            pl.BlockSpec((1, D), lambda i: (0, 0)),
            pl.BlockSpec((D, D), lambda i: (0, 0)),
        ],
        out_specs=pl.BlockSpec((RB, D), lambda i: (i, 0)),
        out_shape=jax.ShapeDtypeStruct((N, D), jnp.float32),
    )(aggr, cnt, x, Wl, bl.reshape(1, D), Wr)


def _gather_kernel(idx_ref, w_ref, xc_ref, o_ref):
    j = jax.lax.broadcasted_iota(jnp.int32, (RB, NP), 1)
    oh = jnp.zeros((RB, NP), jnp.float32)
    for k in range(K):
        oh = oh + jnp.where(j == idx_ref[:, k:k + 1], w_ref[:, k:k + 1], 0.0)
    o_ref[...] = jnp.dot(oh, xc_ref[...], preferred_element_type=jnp.float32)


def _gather(idx, w, xcp):
    return pl.pallas_call(
        _gather_kernel,
        grid=(NBLK,),
        in_specs=[
            pl.BlockSpec((RB, K), lambda i: (i, 0)),
            pl.BlockSpec((RB, K), lambda i: (i, 0)),
            pl.BlockSpec((NP, D), lambda i: (0, 0)),
        ],
        out_specs=pl.BlockSpec((RB, D), lambda i: (i, 0)),
        out_shape=jax.ShapeDtypeStruct((N, D), jnp.float32),
    )(idx, w, xcp)


def _final_kernel(xc_ref, s_ref, wsum_ref, bsum_ref, wbi_ref, bbi_ref,
                  wgu_ref, bgu_ref, wgv_ref, bgv_ref, wgw_ref, bgw_ref,
                  gamma_ref, beta_ref, o_ref):
    xc = xc_ref[...]
    s = s_ref[...]
    dot = lambda a, b: jnp.dot(a, b, preferred_element_type=jnp.float32)
    sum_msg = dot(xc + s, wsum_ref[...]) + bsum_ref[...]
    bi_msg = dot(xc * s, wbi_ref[...]) + bbi_ref[...]
    u = dot(xc, wgu_ref[...]) + bgu_ref[...]
    v = dot(s, wgv_ref[...]) + bgv_ref[...]
    gl = dot(u + v, wgw_ref[...]) + bgw_ref[...]
    g = 1.0 / (1.0 + jnp.exp(-gl))
    y = g * sum_msg + (1.0 - g) * bi_msg
    y = jnp.where(y > 0, y, 0.01 * y)
    res = y + xc
    mu = jnp.mean(res, axis=1, keepdims=True)
    var = jnp.mean((res - mu) ** 2, axis=1, keepdims=True)
    o_ref[...] = (res - mu) / jnp.sqrt(var + 1e-5) * gamma_ref[...] + beta_ref[...]


def _final(xc, summed, W_sum, b_sum, W_bi, b_bi, W_gu, b_gu, W_gv, b_gv,
           W_gw, b_gw, gamma, beta):
    H = D // 2
    return pl.pallas_call(
        _final_kernel,
        grid=(NBLK,),
        in_specs=[
            pl.BlockSpec((RB, D), lambda i: (i, 0)),
            pl.BlockSpec((RB, D), lambda i: (i, 0)),
            pl.BlockSpec((D, D), lambda i: (0, 0)),
            pl.BlockSpec((1, D), lambda i: (0, 0)),
            pl.BlockSpec((D, D), lambda i: (0, 0)),
            pl.BlockSpec((1, D), lambda i: (0, 0)),
            pl.BlockSpec((D, H), lambda i: (0, 0)),
            pl.BlockSpec((1, H), lambda i: (0, 0)),
            pl.BlockSpec((D, H), lambda i: (0, 0)),
            pl.BlockSpec((1, H), lambda i: (0, 0)),
            pl.BlockSpec((H, D), lambda i: (0, 0)),
            pl.BlockSpec((1, D), lambda i: (0, 0)),
            pl.BlockSpec((1, D), lambda i: (0, 0)),
            pl.BlockSpec((1, D), lambda i: (0, 0)),
        ],
        out_specs=pl.BlockSpec((RB, D), lambda i: (i, 0)),
        out_shape=jax.ShapeDtypeStruct((N, D), jnp.float32),
    )(xc, summed, W_sum, b_sum.reshape(1, D), W_bi, b_bi.reshape(1, D),
      W_gu, b_gu.reshape(1, H), W_gv, b_gv.reshape(1, H), W_gw,
      b_gw.reshape(1, D), gamma.reshape(1, D), beta.reshape(1, D))


def kernel(X, W_head, b_head, W_tail, b_tail, Wl0, bl0, Wr0, Wl1, bl1, Wr1,
           W_sum, b_sum, W_bi, b_bi, W_gu, b_gu, W_gv, b_gv, W_gw, b_gw,
           gamma, beta):
    e_h, e_t = _heads(X, W_head, b_head, W_tail, b_tail)
    e_tp = jnp.pad(e_t, ((0, NP - N), (0, 0)))
    weights, topk_idx = _topk(e_h, e_tp)

    src = jnp.repeat(jnp.arange(N, dtype=jnp.int32), K)
    dst = topk_idx.reshape(-1)
    edge_index = jnp.stack([src, dst], axis=0)

    idxt3 = topk_idx.reshape(NBLK, RB, K).transpose(0, 2, 1)

    aggr0, cnt = _scatter(idxt3, X)
    xc0 = _sage_dense(aggr0[:N], cnt[:N], X, Wl0, bl0, Wr0)
    aggr1, _ = _scatter(idxt3, xc0)
    xc1 = _sage_dense(aggr1[:N], cnt[:N], xc0, Wl1, bl1, Wr1)

    xc1p = jnp.pad(xc1, ((0, NP - N), (0, 0)))
    summed = _gather(topk_idx, weights, xc1p)

    out = _final(xc1, summed, W_sum, b_sum, W_bi, b_bi, W_gu, b_gu,
                 W_gv, b_gv, W_gw, b_gw, gamma, beta)
    return (out, edge_index)


# speedup vs baseline: 4.7765x; 1.5011x over previous
import functools

import jax
import jax.numpy as jnp
from jax import lax
from jax.experimental import pallas as pl
from jax.experimental.pallas import tpu as pltpu
from jax.experimental.pallas import tpu_sc as plsc

N = 10000
D = 256
K = 16
NP = 10112
RB = 200
NBLK = N // RB
NEG = -1.0e30

NSC = 2
NSUB = 16
NPAD = 10240
RT = NPAD // NSUB
DH = D // 2
DUMP = N + 16


def _mm_kernel(x_ref, wh_ref, bh_ref, wt_ref, bt_ref, eh_ref, et_ref):
    x = x_ref[...]
    eh_ref[...] = jnp.dot(x, wh_ref[...], preferred_element_type=jnp.float32) + bh_ref[...]
    et_ref[...] = jnp.dot(x, wt_ref[...], preferred_element_type=jnp.float32) + bt_ref[...]


def _heads(X, W_head, b_head, W_tail, b_tail):
    return pl.pallas_call(
        _mm_kernel,
        grid=(NBLK,),
        in_specs=[
            pl.BlockSpec((RB, D), lambda i: (i, 0)),
            pl.BlockSpec((D, D), lambda i: (0, 0)),
            pl.BlockSpec((1, D), lambda i: (0, 0)),
            pl.BlockSpec((D, D), lambda i: (0, 0)),
            pl.BlockSpec((1, D), lambda i: (0, 0)),
        ],
        out_specs=[
            pl.BlockSpec((RB, D), lambda i: (i, 0)),
            pl.BlockSpec((RB, D), lambda i: (i, 0)),
        ],
        out_shape=[
            jax.ShapeDtypeStruct((N, D), jnp.float32),
            jax.ShapeDtypeStruct((N, D), jnp.float32),
        ],
    )(X, W_head, b_head.reshape(1, D), W_tail, b_tail.reshape(1, D))


def _topk_kernel(eh_ref, etp_ref, w_ref, idx_ref):
    scale = D ** (-0.5)
    logits = jax.lax.dot_general(
        eh_ref[...], etp_ref[...], (((1,), (1,)), ((), ())),
        preferred_element_type=jnp.float32) * scale
    col = jax.lax.broadcasted_iota(jnp.int32, (RB, NP), 1)
    logits = jnp.where(col < N, logits, NEG)
    vals, idxs = [], []
    l = logits
    for _ in range(K):
        m = jnp.max(l, axis=1, keepdims=True)
        am = jnp.min(jnp.where(l == m, col, jnp.int32(NP)), axis=1, keepdims=True)
        vals.append(m)
        idxs.append(am)
        l = jnp.where(col == am, NEG, l)
    v = jnp.concatenate(vals, axis=1)
    ix = jnp.concatenate(idxs, axis=1)
    e = jnp.exp(v - v[:, 0:1])
    w_ref[...] = e / jnp.sum(e, axis=1, keepdims=True)
    idx_ref[...] = ix


def _topk(e_h, e_tp):
    return pl.pallas_call(
        _topk_kernel,
        grid=(NBLK,),
        in_specs=[
            pl.BlockSpec((RB, D), lambda i: (i, 0)),
            pl.BlockSpec((NP, D), lambda i: (0, 0)),
        ],
        out_specs=[
            pl.BlockSpec((RB, K), lambda i: (i, 0)),
            pl.BlockSpec((RB, K), lambda i: (i, 0)),
        ],
        out_shape=[
            jax.ShapeDtypeStruct((N, K), jnp.float32),
            jax.ShapeDtypeStruct((N, K), jnp.int32),
        ],
    )(e_h, e_tp)


def _sc_scatter_body(xh_ref, idxt_ref, zro_ref, aggr_ref, xv, idxv, agg_sh):
    c = lax.axis_index("c")
    s = lax.axis_index("s")
    rows = pl.ds(RT * s, RT)
    pltpu.sync_copy(zro_ref.at[rows], agg_sh.at[rows])
    pltpu.sync_copy(idxt_ref.at[s], idxv)
    plsc.subcore_barrier()

    def _scat_ch(ch, _):
        pltpu.sync_copy(xh_ref.at[pl.ds(NPAD * c + RT * s + 128 * ch, 128)],
                        xv)

        def _scat_k(k, _):
            pltpu.sync_copy(xv, agg_sh.at[idxv.at[k, ch]], add=True)
            return 0

        lax.fori_loop(0, K, _scat_k, 0)
        return 0

    lax.fori_loop(0, RT // 128, _scat_ch, 0)

    plsc.subcore_barrier()
    pltpu.sync_copy(agg_sh.at[rows], aggr_ref.at[pl.ds(NPAD * c + RT * s, RT)])


@functools.cache
def _get_sc_scatter():
    return pl.kernel(
        _sc_scatter_body,
        out_type=jax.ShapeDtypeStruct((NSC * NPAD, DH), jnp.float32),
        mesh=plsc.VectorSubcoreMesh(core_axis_name="c", subcore_axis_name="s"),
        scratch_types=[
            pltpu.VMEM((128, DH), jnp.float32),
            pltpu.VMEM((K, RT // 128, 128), jnp.int32),
            pltpu.VMEM_SHARED((NPAD, DH), jnp.float32),
        ],
    )


NCHUNK = (RT * K) // 128


def _sc_gather_body(xc2_ref, idxf_ref, g_ref, idxv, gbuf, sem):
    c = lax.axis_index("c")
    s = lax.axis_index("s")
    w = c * NSUB + s
    pltpu.sync_copy(idxf_ref.at[w], idxv)
    base = c * NPAD * K + s * RT * K

    def _gath(ch, _):
        pltpu.async_copy(xc2_ref.at[idxv.at[ch]], gbuf, sem).wait()
        pltpu.sync_copy(gbuf, g_ref.at[pl.ds(base + ch * 128, 128)])
        return 0

    lax.fori_loop(0, NCHUNK, _gath, 0)


@functools.cache
def _get_sc_gather():
    return pl.kernel(
        _sc_gather_body,
        out_type=jax.ShapeDtypeStruct((NSC * NPAD * K, DH), jnp.float32),
        mesh=plsc.VectorSubcoreMesh(core_axis_name="c", subcore_axis_name="s"),
        scratch_types=[
            pltpu.VMEM((RT * K // 128, 128), jnp.int32),
            pltpu.VMEM((128, DH), jnp.float32),
            pltpu.SemaphoreType.DMA,
        ],
    )


def _sage_dense_kernel(a0_ref, a1_ref, cnt_ref, x_ref, wl_ref, bl_ref, wr_ref,
                       o_ref):
    aggr = jnp.concatenate([a0_ref[...], a1_ref[...]], axis=1)
    aggr = aggr / jnp.maximum(cnt_ref[...], 1.0)
    y = (jnp.dot(aggr, wl_ref[...], preferred_element_type=jnp.float32)
         + bl_ref[...]
         + jnp.dot(x_ref[...], wr_ref[...], preferred_element_type=jnp.float32))
    o_ref[...] = jnp.where(y > 0, y, 0.01 * y)


def _sage_dense(a0, a1, cnt, x, Wl, bl, Wr):
    return pl.pallas_call(
        _sage_dense_kernel,
        grid=(NBLK,),
        in_specs=[
            pl.BlockSpec((RB, DH), lambda i: (i, 0)),
            pl.BlockSpec((RB, DH), lambda i: (i, 0)),
            pl.BlockSpec((RB, 1), lambda i: (i, 0)),
            pl.BlockSpec((RB, D), lambda i: (i, 0)),
            pl.BlockSpec((D, D), lambda i: (0, 0)),
            pl.BlockSpec((1, D), lambda i: (0, 0)),
            pl.BlockSpec((D, D), lambda i: (0, 0)),
        ],
        out_specs=pl.BlockSpec((RB, D), lambda i: (i, 0)),
        out_shape=jax.ShapeDtypeStruct((N, D), jnp.float32),
    )(a0, a1, cnt, x, Wl, bl.reshape(1, D), Wr)


def _final_kernel(xc_ref, w_ref, g0_ref, g1_ref, wsum_ref, bsum_ref, wbi_ref,
                  bbi_ref, wgu_ref, bgu_ref, wgv_ref, bgv_ref, wgw_ref,
                  bgw_ref, gamma_ref, beta_ref, o_ref):
    xc = xc_ref[...]
    s = jnp.zeros((RB, D), jnp.float32)
    for k in range(K):
        gk = jnp.concatenate([g0_ref[:, k, :], g1_ref[:, k, :]], axis=1)
        s = s + w_ref[:, k:k + 1] * gk
    dot = lambda a, b: jnp.dot(a, b, preferred_element_type=jnp.float32)
    sum_msg = dot(xc + s, wsum_ref[...]) + bsum_ref[...]
    bi_msg = dot(xc * s, wbi_ref[...]) + bbi_ref[...]
    u = dot(xc, wgu_ref[...]) + bgu_ref[...]
    v = dot(s, wgv_ref[...]) + bgv_ref[...]
    gl = dot(u + v, wgw_ref[...]) + bgw_ref[...]
    g = 1.0 / (1.0 + jnp.exp(-gl))
    y = g * sum_msg + (1.0 - g) * bi_msg
    y = jnp.where(y > 0, y, 0.01 * y)
    res = y + xc
    mu = jnp.mean(res, axis=1, keepdims=True)
    var = jnp.mean((res - mu) ** 2, axis=1, keepdims=True)
    o_ref[...] = (res - mu) / jnp.sqrt(var + 1e-5) * gamma_ref[...] + beta_ref[...]


def _final(xc, w, g0, g1, W_sum, b_sum, W_bi, b_bi, W_gu, b_gu, W_gv, b_gv,
           W_gw, b_gw, gamma, beta):
    H = D // 2
    return pl.pallas_call(
        _final_kernel,
        grid=(NBLK,),
        in_specs=[
            pl.BlockSpec((RB, D), lambda i: (i, 0)),
            pl.BlockSpec((RB, K), lambda i: (i, 0)),
            pl.BlockSpec((RB, K, DH), lambda i: (i, 0, 0)),
            pl.BlockSpec((RB, K, DH), lambda i: (i, 0, 0)),
            pl.BlockSpec((D, D), lambda i: (0, 0)),
            pl.BlockSpec((1, D), lambda i: (0, 0)),
            pl.BlockSpec((D, D), lambda i: (0, 0)),
            pl.BlockSpec((1, D), lambda i: (0, 0)),
            pl.BlockSpec((D, H), lambda i: (0, 0)),
            pl.BlockSpec((1, H), lambda i: (0, 0)),
            pl.BlockSpec((D, H), lambda i: (0, 0)),
            pl.BlockSpec((1, H), lambda i: (0, 0)),
            pl.BlockSpec((H, D), lambda i: (0, 0)),
            pl.BlockSpec((1, D), lambda i: (0, 0)),
            pl.BlockSpec((1, D), lambda i: (0, 0)),
            pl.BlockSpec((1, D), lambda i: (0, 0)),
        ],
        out_specs=pl.BlockSpec((RB, D), lambda i: (i, 0)),
        out_shape=jax.ShapeDtypeStruct((N, D), jnp.float32),
    )(xc, w, g0, g1, W_sum, b_sum.reshape(1, D), W_bi, b_bi.reshape(1, D),
      W_gu, b_gu.reshape(1, H), W_gv, b_gv.reshape(1, H), W_gw,
      b_gw.reshape(1, D), gamma.reshape(1, D), beta.reshape(1, D))


def _split_pad(x):
    xp = jnp.pad(x, ((0, NPAD - N), (0, 0)))
    return jnp.concatenate([xp[:, :DH], xp[:, DH:]], axis=0)


def kernel(X, W_head, b_head, W_tail, b_tail, Wl0, bl0, Wr0, Wl1, bl1, Wr1,
           W_sum, b_sum, W_bi, b_bi, W_gu, b_gu, W_gv, b_gv, W_gw, b_gw,
           gamma, beta):
    e_h, e_t = _heads(X, W_head, b_head, W_tail, b_tail)
    e_tp = jnp.pad(e_t, ((0, NP - N), (0, 0)))
    weights, topk_idx = _topk(e_h, e_tp)

    src = jnp.repeat(jnp.arange(N, dtype=jnp.int32), K)
    dst = topk_idx.reshape(-1)
    edge_index = jnp.stack([src, dst], axis=0)

    idxp = jnp.concatenate(
        [topk_idx, jnp.full((NPAD - N, K), DUMP, jnp.int32)], axis=0)
    idxt = idxp.reshape(NSUB, RT, K).transpose(0, 2, 1).reshape(
        NSUB, K, RT // 128, 128)
    per_tile = idxp.reshape(-1).reshape(NSUB, RT * K)
    idxf = jnp.concatenate([per_tile, per_tile + NPAD], axis=0).reshape(
        NSC * NSUB, RT * K // 128, 128)

    zro = jnp.zeros((NPAD, DH), jnp.float32)

    onesf = jnp.ones((NSC * NPAD, DH), jnp.float32)
    cnt = _get_sc_scatter()(onesf, idxt, zro)[:NPAD, 0:1]
    aggr0 = _get_sc_scatter()(_split_pad(X), idxt, zro)
    xc0 = _sage_dense(aggr0[:NPAD], aggr0[NPAD:], cnt, X, Wl0, bl0, Wr0)
    aggr1 = _get_sc_scatter()(_split_pad(xc0), idxt, zro)
    xc1 = _sage_dense(aggr1[:NPAD], aggr1[NPAD:], cnt, xc0, Wl1, bl1, Wr1)

    G = _get_sc_gather()(_split_pad(xc1), idxf)
    g0 = G[:NPAD * K].reshape(NPAD, K, DH)
    g1 = G[NPAD * K:].reshape(NPAD, K, DH)

    out = _final(xc1, weights, g0, g1, W_sum, b_sum, W_bi, b_bi, W_gu, b_gu,
                 W_gv, b_gv, W_gw, b_gw, gamma, beta)
    return (out, edge_index)


# speedup vs baseline: 5.3647x; 1.1232x over previous
import functools

import jax
import jax.numpy as jnp
from jax import lax
from jax.experimental import pallas as pl
from jax.experimental.pallas import tpu as pltpu
from jax.experimental.pallas import tpu_sc as plsc

N = 10000
D = 256
K = 16
NP = 10112
RB = 200
NBLK = N // RB
NEG = -1.0e30

NSC = 2
NSUB = 16
NPAD = 10240
RT = NPAD // NSUB
DH = D // 2
DUMP = N + 16


def _mm_kernel(x_ref, wh_ref, bh_ref, wt_ref, bt_ref, eh_ref, et_ref):
    x = x_ref[...]
    eh_ref[...] = jnp.dot(x, wh_ref[...], preferred_element_type=jnp.float32) + bh_ref[...]
    et_ref[...] = jnp.dot(x, wt_ref[...], preferred_element_type=jnp.float32) + bt_ref[...]


def _heads(X, W_head, b_head, W_tail, b_tail):
    return pl.pallas_call(
        _mm_kernel,
        grid=(NBLK,),
        in_specs=[
            pl.BlockSpec((RB, D), lambda i: (i, 0)),
            pl.BlockSpec((D, D), lambda i: (0, 0)),
            pl.BlockSpec((1, D), lambda i: (0, 0)),
            pl.BlockSpec((D, D), lambda i: (0, 0)),
            pl.BlockSpec((1, D), lambda i: (0, 0)),
        ],
        out_specs=[
            pl.BlockSpec((RB, D), lambda i: (i, 0)),
            pl.BlockSpec((RB, D), lambda i: (i, 0)),
        ],
        out_shape=[
            jax.ShapeDtypeStruct((N, D), jnp.float32),
            jax.ShapeDtypeStruct((N, D), jnp.float32),
        ],
    )(X, W_head, b_head.reshape(1, D), W_tail, b_tail.reshape(1, D))


def _topk_kernel(eh_ref, etp_ref, w_ref, idx_ref):
    scale = D ** (-0.5)
    logits = jax.lax.dot_general(
        eh_ref[...], etp_ref[...], (((1,), (1,)), ((), ())),
        preferred_element_type=jnp.float32) * scale
    col = jax.lax.broadcasted_iota(jnp.int32, (RB, NP), 1)
    logits = jnp.where(col < N, logits, NEG)
    vals, idxs = [], []
    l = logits
    for _ in range(K):
        m = jnp.max(l, axis=1, keepdims=True)
        eq = l == m
        am = jnp.min(jnp.where(eq, col, jnp.int32(NP)), axis=1, keepdims=True)
        vals.append(m)
        idxs.append(am)
        l = jnp.where(eq, NEG, l)
    v = jnp.concatenate(vals, axis=1)
    ix = jnp.concatenate(idxs, axis=1)
    e = jnp.exp(v - v[:, 0:1])
    w_ref[...] = e / jnp.sum(e, axis=1, keepdims=True)
    idx_ref[...] = ix


def _topk(e_h, e_tp):
    return pl.pallas_call(
        _topk_kernel,
        grid=(NBLK,),
        in_specs=[
            pl.BlockSpec((RB, D), lambda i: (i, 0)),
            pl.BlockSpec((NP, D), lambda i: (0, 0)),
        ],
        out_specs=[
            pl.BlockSpec((RB, K), lambda i: (i, 0)),
            pl.BlockSpec((RB, K), lambda i: (i, 0)),
        ],
        out_shape=[
            jax.ShapeDtypeStruct((N, K), jnp.float32),
            jax.ShapeDtypeStruct((N, K), jnp.int32),
        ],
    )(e_h, e_tp)


def _sc_scatter_body(xh_ref, idxt_ref, zro_ref, aggr_ref, xv, idxv, agg_sh):
    c = lax.axis_index("c")
    s = lax.axis_index("s")
    rows = pl.ds(RT * s, RT)
    pltpu.sync_copy(zro_ref.at[rows], agg_sh.at[rows])
    pltpu.sync_copy(idxt_ref.at[s], idxv)
    plsc.subcore_barrier()

    def _scat_ch(ch, _):
        pltpu.sync_copy(xh_ref.at[pl.ds(NPAD * c + RT * s + 128 * ch, 128)],
                        xv)

        def _scat_k(k, _):
            pltpu.sync_copy(xv, agg_sh.at[idxv.at[k, ch]], add=True)
            return 0

        lax.fori_loop(0, K, _scat_k, 0)
        return 0

    lax.fori_loop(0, RT // 128, _scat_ch, 0)

    plsc.subcore_barrier()
    pltpu.sync_copy(agg_sh.at[rows], aggr_ref.at[pl.ds(NPAD * c + RT * s, RT)])


@functools.cache
def _get_sc_scatter():
    return pl.kernel(
        _sc_scatter_body,
        out_type=jax.ShapeDtypeStruct((NSC * NPAD, DH), jnp.float32),
        mesh=plsc.VectorSubcoreMesh(core_axis_name="c", subcore_axis_name="s"),
        scratch_types=[
            pltpu.VMEM((128, DH), jnp.float32),
            pltpu.VMEM((K, RT // 128, 128), jnp.int32),
            pltpu.VMEM_SHARED((NPAD, DH), jnp.float32),
        ],
    )


NCHUNK = (RT * K) // 128


def _sc_gather_body(xc2_ref, idxf_ref, g_ref, idxv, gbuf, sems):
    c = lax.axis_index("c")
    s = lax.axis_index("s")
    w = c * NSUB + s
    pltpu.sync_copy(idxf_ref.at[w], idxv)
    base = c * NPAD * K + s * RT * K

    pltpu.async_copy(xc2_ref.at[idxv.at[0]], gbuf.at[pl.ds(0, 128)],
                     sems.at[0])

    def _gath(ch, _):
        par = lax.rem(ch, 2)
        nxt = lax.rem(ch + 1, 2)

        @pl.when(ch + 1 < NCHUNK)
        def _():
            pltpu.async_copy(xc2_ref.at[idxv.at[ch + 1]],
                             gbuf.at[pl.ds(128 * nxt, 128)], sems.at[nxt])

        pltpu.make_async_copy(
            xc2_ref.at[idxv.at[ch]], gbuf.at[pl.ds(128 * par, 128)],
            sems.at[par]).wait()
        pltpu.sync_copy(gbuf.at[pl.ds(128 * par, 128)],
                        g_ref.at[pl.ds(base + ch * 128, 128)])
        return 0

    lax.fori_loop(0, NCHUNK, _gath, 0)


@functools.cache
def _get_sc_gather():
    return pl.kernel(
        _sc_gather_body,
        out_type=jax.ShapeDtypeStruct((NSC * NPAD * K, DH), jnp.float32),
        mesh=plsc.VectorSubcoreMesh(core_axis_name="c", subcore_axis_name="s"),
        scratch_types=[
            pltpu.VMEM((RT * K // 128, 128), jnp.int32),
            pltpu.VMEM((256, DH), jnp.float32),
            pltpu.SemaphoreType.DMA((2,)),
        ],
    )


def _sage_dense_kernel(a0_ref, a1_ref, cnt_ref, x_ref, wl_ref, bl_ref, wr_ref,
                       o_ref):
    aggr = jnp.concatenate([a0_ref[...], a1_ref[...]], axis=1)
    aggr = aggr / jnp.maximum(cnt_ref[...], 1.0)
    y = (jnp.dot(aggr, wl_ref[...], preferred_element_type=jnp.float32)
         + bl_ref[...]
         + jnp.dot(x_ref[...], wr_ref[...], preferred_element_type=jnp.float32))
    o_ref[...] = jnp.where(y > 0, y, 0.01 * y)


def _sage_dense(a0, a1, cnt, x, Wl, bl, Wr):
    return pl.pallas_call(
        _sage_dense_kernel,
        grid=(NBLK,),
        in_specs=[
            pl.BlockSpec((RB, DH), lambda i: (i, 0)),
            pl.BlockSpec((RB, DH), lambda i: (i, 0)),
            pl.BlockSpec((RB, 1), lambda i: (i, 0)),
            pl.BlockSpec((RB, D), lambda i: (i, 0)),
            pl.BlockSpec((D, D), lambda i: (0, 0)),
            pl.BlockSpec((1, D), lambda i: (0, 0)),
            pl.BlockSpec((D, D), lambda i: (0, 0)),
        ],
        out_specs=pl.BlockSpec((RB, D), lambda i: (i, 0)),
        out_shape=jax.ShapeDtypeStruct((N, D), jnp.float32),
    )(a0, a1, cnt, x, Wl, bl.reshape(1, D), Wr)


def _final_kernel(xc_ref, w_ref, g0_ref, g1_ref, wsum_ref, bsum_ref, wbi_ref,
                  bbi_ref, wgu_ref, bgu_ref, wgv_ref, bgv_ref, wgw_ref,
                  bgw_ref, gamma_ref, beta_ref, o_ref):
    xc = xc_ref[...]
    s = jnp.zeros((RB, D), jnp.float32)
    for k in range(K):
        gk = jnp.concatenate([g0_ref[:, k, :], g1_ref[:, k, :]], axis=1)
        s = s + w_ref[:, k:k + 1] * gk
    dot = lambda a, b: jnp.dot(a, b, preferred_element_type=jnp.float32)
    sum_msg = dot(xc + s, wsum_ref[...]) + bsum_ref[...]
    bi_msg = dot(xc * s, wbi_ref[...]) + bbi_ref[...]
    u = dot(xc, wgu_ref[...]) + bgu_ref[...]
    v = dot(s, wgv_ref[...]) + bgv_ref[...]
    gl = dot(u + v, wgw_ref[...]) + bgw_ref[...]
    g = 1.0 / (1.0 + jnp.exp(-gl))
    y = g * sum_msg + (1.0 - g) * bi_msg
    y = jnp.where(y > 0, y, 0.01 * y)
    res = y + xc
    mu = jnp.mean(res, axis=1, keepdims=True)
    var = jnp.mean((res - mu) ** 2, axis=1, keepdims=True)
    o_ref[...] = (res - mu) / jnp.sqrt(var + 1e-5) * gamma_ref[...] + beta_ref[...]


def _final(xc, w, g0, g1, W_sum, b_sum, W_bi, b_bi, W_gu, b_gu, W_gv, b_gv,
           W_gw, b_gw, gamma, beta):
    H = D // 2
    return pl.pallas_call(
        _final_kernel,
        grid=(NBLK,),
        in_specs=[
            pl.BlockSpec((RB, D), lambda i: (i, 0)),
            pl.BlockSpec((RB, K), lambda i: (i, 0)),
            pl.BlockSpec((RB, K, DH), lambda i: (i, 0, 0)),
            pl.BlockSpec((RB, K, DH), lambda i: (i, 0, 0)),
            pl.BlockSpec((D, D), lambda i: (0, 0)),
            pl.BlockSpec((1, D), lambda i: (0, 0)),
            pl.BlockSpec((D, D), lambda i: (0, 0)),
            pl.BlockSpec((1, D), lambda i: (0, 0)),
            pl.BlockSpec((D, H), lambda i: (0, 0)),
            pl.BlockSpec((1, H), lambda i: (0, 0)),
            pl.BlockSpec((D, H), lambda i: (0, 0)),
            pl.BlockSpec((1, H), lambda i: (0, 0)),
            pl.BlockSpec((H, D), lambda i: (0, 0)),
            pl.BlockSpec((1, D), lambda i: (0, 0)),
            pl.BlockSpec((1, D), lambda i: (0, 0)),
            pl.BlockSpec((1, D), lambda i: (0, 0)),
        ],
        out_specs=pl.BlockSpec((RB, D), lambda i: (i, 0)),
        out_shape=jax.ShapeDtypeStruct((N, D), jnp.float32),
    )(xc, w, g0, g1, W_sum, b_sum.reshape(1, D), W_bi, b_bi.reshape(1, D),
      W_gu, b_gu.reshape(1, H), W_gv, b_gv.reshape(1, H), W_gw,
      b_gw.reshape(1, D), gamma.reshape(1, D), beta.reshape(1, D))


def _split_pad(x):
    xp = jnp.pad(x, ((0, NPAD - N), (0, 0)))
    return jnp.concatenate([xp[:, :DH], xp[:, DH:]], axis=0)


def kernel(X, W_head, b_head, W_tail, b_tail, Wl0, bl0, Wr0, Wl1, bl1, Wr1,
           W_sum, b_sum, W_bi, b_bi, W_gu, b_gu, W_gv, b_gv, W_gw, b_gw,
           gamma, beta):
    e_h, e_t = _heads(X, W_head, b_head, W_tail, b_tail)
    e_tp = jnp.pad(e_t, ((0, NP - N), (0, 0)))
    weights, topk_idx = _topk(e_h, e_tp)

    src = jnp.repeat(jnp.arange(N, dtype=jnp.int32), K)
    dst = topk_idx.reshape(-1)
    edge_index = jnp.stack([src, dst], axis=0)

    idxp = jnp.concatenate(
        [topk_idx, jnp.full((NPAD - N, K), DUMP, jnp.int32)], axis=0)
    idxt = idxp.reshape(NSUB, RT, K).transpose(0, 2, 1).reshape(
        NSUB, K, RT // 128, 128)
    per_tile = idxp.reshape(-1).reshape(NSUB, RT * K)
    idxf = jnp.concatenate([per_tile, per_tile + NPAD], axis=0).reshape(
        NSC * NSUB, RT * K // 128, 128)

    zro = jnp.zeros((NPAD, DH), jnp.float32)

    onesf = jnp.ones((NSC * NPAD, DH), jnp.float32)
    cnt = _get_sc_scatter()(onesf, idxt, zro)[:NPAD, 0:1]
    aggr0 = _get_sc_scatter()(_split_pad(X), idxt, zro)
    xc0 = _sage_dense(aggr0[:NPAD], aggr0[NPAD:], cnt, X, Wl0, bl0, Wr0)
    aggr1 = _get_sc_scatter()(_split_pad(xc0), idxt, zro)
    xc1 = _sage_dense(aggr1[:NPAD], aggr1[NPAD:], cnt, xc0, Wl1, bl1, Wr1)

    G = _get_sc_gather()(_split_pad(xc1), idxf)
    g0 = G[:NPAD * K].reshape(NPAD, K, DH)
    g1 = G[NPAD * K:].reshape(NPAD, K, DH)

    out = _final(xc1, weights, g0, g1, W_sum, b_sum, W_bi, b_bi, W_gu, b_gu,
                 W_gv, b_gv, W_gw, b_gw, gamma, beta)
    return (out, edge_index)


# speedup vs baseline: 5.9528x; 1.1096x over previous
import functools

import jax
import jax.numpy as jnp
from jax import lax
from jax.experimental import pallas as pl
from jax.experimental.pallas import tpu as pltpu
from jax.experimental.pallas import tpu_sc as plsc

N = 10000
D = 256
K = 16
NP = 10112
RB = 200
NBLK = N // RB
NEG = -1.0e30

NSC = 2
NSUB = 16
NPAD = 10240
RT = NPAD // NSUB
DH = D // 2
DUMP = N + 16


def _mm_kernel(x_ref, wh_ref, bh_ref, wt_ref, bt_ref, eh_ref, et_ref):
    x = x_ref[...]
    eh_ref[...] = jnp.dot(x, wh_ref[...], preferred_element_type=jnp.float32) + bh_ref[...]
    et_ref[...] = jnp.dot(x, wt_ref[...], preferred_element_type=jnp.float32) + bt_ref[...]


def _heads(X, W_head, b_head, W_tail, b_tail):
    return pl.pallas_call(
        _mm_kernel,
        grid=(NBLK,),
        in_specs=[
            pl.BlockSpec((RB, D), lambda i: (i, 0)),
            pl.BlockSpec((D, D), lambda i: (0, 0)),
            pl.BlockSpec((1, D), lambda i: (0, 0)),
            pl.BlockSpec((D, D), lambda i: (0, 0)),
            pl.BlockSpec((1, D), lambda i: (0, 0)),
        ],
        out_specs=[
            pl.BlockSpec((RB, D), lambda i: (i, 0)),
            pl.BlockSpec((RB, D), lambda i: (i, 0)),
        ],
        out_shape=[
            jax.ShapeDtypeStruct((N, D), jnp.float32),
            jax.ShapeDtypeStruct((N, D), jnp.float32),
        ],
    )(X, W_head, b_head.reshape(1, D), W_tail, b_tail.reshape(1, D))


def _topk_kernel(eh_ref, etp_ref, w_ref, idx_ref):
    scale = D ** (-0.5)
    logits = jax.lax.dot_general(
        eh_ref[...], etp_ref[...], (((1,), (1,)), ((), ())),
        preferred_element_type=jnp.float32) * scale
    col = jax.lax.broadcasted_iota(jnp.int32, (RB, NP), 1)
    logits = jnp.where(col < N, logits, NEG)
    vals, idxs = [], []
    l = logits
    for _ in range(K):
        m = jnp.max(l, axis=1, keepdims=True)
        eq = l == m
        am = jnp.min(jnp.where(eq, col, jnp.int32(NP)), axis=1, keepdims=True)
        vals.append(m)
        idxs.append(am)
        l = jnp.where(eq, NEG, l)
    v = jnp.concatenate(vals, axis=1)
    ix = jnp.concatenate(idxs, axis=1)
    e = jnp.exp(v - v[:, 0:1])
    w_ref[...] = e / jnp.sum(e, axis=1, keepdims=True)
    idx_ref[...] = ix


def _topk(e_h, e_tp):
    return pl.pallas_call(
        _topk_kernel,
        grid=(NBLK,),
        in_specs=[
            pl.BlockSpec((RB, D), lambda i: (i, 0)),
            pl.BlockSpec((NP, D), lambda i: (0, 0)),
        ],
        out_specs=[
            pl.BlockSpec((RB, K), lambda i: (i, 0)),
            pl.BlockSpec((RB, K), lambda i: (i, 0)),
        ],
        out_shape=[
            jax.ShapeDtypeStruct((N, K), jnp.float32),
            jax.ShapeDtypeStruct((N, K), jnp.int32),
        ],
    )(e_h, e_tp)


def _sc_scatter_body(xh_ref, idxt_ref, zro_ref, aggr_ref, xv, idxv, agg_sh):
    c = lax.axis_index("c")
    s = lax.axis_index("s")
    rows = pl.ds(RT * s, RT)
    pltpu.sync_copy(zro_ref.at[rows], agg_sh.at[rows])
    pltpu.sync_copy(idxt_ref.at[s], idxv)
    plsc.subcore_barrier()

    def _scat_ch(ch, _):
        pltpu.sync_copy(xh_ref.at[pl.ds(NPAD * c + RT * s + 128 * ch, 128)],
                        xv)

        def _scat_k(k, _):
            pltpu.sync_copy(xv, agg_sh.at[idxv.at[k, ch]], add=True)
            return 0

        lax.fori_loop(0, K, _scat_k, 0)
        return 0

    lax.fori_loop(0, RT // 128, _scat_ch, 0)

    plsc.subcore_barrier()
    pltpu.sync_copy(agg_sh.at[rows], aggr_ref.at[pl.ds(NPAD * c + RT * s, RT)])


@functools.cache
def _get_sc_scatter():
    return pl.kernel(
        _sc_scatter_body,
        out_type=jax.ShapeDtypeStruct((NSC * NPAD, DH), jnp.float32),
        mesh=plsc.VectorSubcoreMesh(core_axis_name="c", subcore_axis_name="s"),
        scratch_types=[
            pltpu.VMEM((128, DH), jnp.float32),
            pltpu.VMEM((K, RT // 128, 128), jnp.int32),
            pltpu.VMEM_SHARED((NPAD, DH), jnp.float32),
        ],
    )


NCHUNK = (RT * K) // 128


def _sc_gather_body(xc2_ref, idxf_ref, g_ref, idxv, gbuf, xc_sh, sem):
    c = lax.axis_index("c")
    s = lax.axis_index("s")
    rows = pl.ds(RT * s, RT)
    pltpu.sync_copy(xc2_ref.at[pl.ds(c * NPAD + RT * s, RT)], xc_sh.at[rows])
    plsc.subcore_barrier()
    base = c * NPAD * K + s * RT * K

    def _half(h, _):
        pltpu.sync_copy(idxf_ref.at[s * 2 + h], idxv)

        def _gath(chh, _):
            pltpu.async_copy(xc_sh.at[idxv.at[chh]], gbuf, sem).wait()
            pltpu.sync_copy(
                gbuf,
                g_ref.at[pl.ds(base + (h * 40 + chh) * 128, 128)])
            return 0

        lax.fori_loop(0, 40, _gath, 0)
        return 0

    lax.fori_loop(0, 2, _half, 0)


@functools.cache
def _get_sc_gather():
    return pl.kernel(
        _sc_gather_body,
        out_type=jax.ShapeDtypeStruct((NSC * NPAD * K, DH), jnp.float32),
        mesh=plsc.VectorSubcoreMesh(core_axis_name="c", subcore_axis_name="s"),
        scratch_types=[
            pltpu.VMEM((40, 128), jnp.int32),
            pltpu.VMEM((128, DH), jnp.float32),
            pltpu.VMEM_SHARED((NPAD, DH), jnp.float32),
            pltpu.SemaphoreType.DMA,
        ],
    )


def _sage_dense_kernel(a0_ref, a1_ref, cnt_ref, x_ref, wl_ref, bl_ref, wr_ref,
                       o_ref):
    aggr = jnp.concatenate([a0_ref[...], a1_ref[...]], axis=1)
    aggr = aggr / jnp.maximum(cnt_ref[...], 1.0)
    y = (jnp.dot(aggr, wl_ref[...], preferred_element_type=jnp.float32)
         + bl_ref[...]
         + jnp.dot(x_ref[...], wr_ref[...], preferred_element_type=jnp.float32))
    o_ref[...] = jnp.where(y > 0, y, 0.01 * y)


def _sage_dense(a0, a1, cnt, x, Wl, bl, Wr):
    return pl.pallas_call(
        _sage_dense_kernel,
        grid=(NBLK,),
        in_specs=[
            pl.BlockSpec((RB, DH), lambda i: (i, 0)),
            pl.BlockSpec((RB, DH), lambda i: (i, 0)),
            pl.BlockSpec((RB, 1), lambda i: (i, 0)),
            pl.BlockSpec((RB, D), lambda i: (i, 0)),
            pl.BlockSpec((D, D), lambda i: (0, 0)),
            pl.BlockSpec((1, D), lambda i: (0, 0)),
            pl.BlockSpec((D, D), lambda i: (0, 0)),
        ],
        out_specs=pl.BlockSpec((RB, D), lambda i: (i, 0)),
        out_shape=jax.ShapeDtypeStruct((N, D), jnp.float32),
    )(a0, a1, cnt, x, Wl, bl.reshape(1, D), Wr)


def _final_kernel(xc_ref, w_ref, g0_ref, g1_ref, wsum_ref, bsum_ref, wbi_ref,
                  bbi_ref, wgu_ref, bgu_ref, wgv_ref, bgv_ref, wgw_ref,
                  bgw_ref, gamma_ref, beta_ref, o_ref):
    xc = xc_ref[...]
    s = jnp.zeros((RB, D), jnp.float32)
    for k in range(K):
        gk = jnp.concatenate([g0_ref[:, k, :], g1_ref[:, k, :]], axis=1)
        s = s + w_ref[:, k:k + 1] * gk
    dot = lambda a, b: jnp.dot(a, b, preferred_element_type=jnp.float32)
    sum_msg = dot(xc + s, wsum_ref[...]) + bsum_ref[...]
    bi_msg = dot(xc * s, wbi_ref[...]) + bbi_ref[...]
    u = dot(xc, wgu_ref[...]) + bgu_ref[...]
    v = dot(s, wgv_ref[...]) + bgv_ref[...]
    gl = dot(u + v, wgw_ref[...]) + bgw_ref[...]
    g = 1.0 / (1.0 + jnp.exp(-gl))
    y = g * sum_msg + (1.0 - g) * bi_msg
    y = jnp.where(y > 0, y, 0.01 * y)
    res = y + xc
    mu = jnp.mean(res, axis=1, keepdims=True)
    var = jnp.mean((res - mu) ** 2, axis=1, keepdims=True)
    o_ref[...] = (res - mu) / jnp.sqrt(var + 1e-5) * gamma_ref[...] + beta_ref[...]


def _final(xc, w, g0, g1, W_sum, b_sum, W_bi, b_bi, W_gu, b_gu, W_gv, b_gv,
           W_gw, b_gw, gamma, beta):
    H = D // 2
    return pl.pallas_call(
        _final_kernel,
        grid=(NBLK,),
        in_specs=[
            pl.BlockSpec((RB, D), lambda i: (i, 0)),
            pl.BlockSpec((RB, K), lambda i: (i, 0)),
            pl.BlockSpec((RB, K, DH), lambda i: (i, 0, 0)),
            pl.BlockSpec((RB, K, DH), lambda i: (i, 0, 0)),
            pl.BlockSpec((D, D), lambda i: (0, 0)),
            pl.BlockSpec((1, D), lambda i: (0, 0)),
            pl.BlockSpec((D, D), lambda i: (0, 0)),
            pl.BlockSpec((1, D), lambda i: (0, 0)),
            pl.BlockSpec((D, H), lambda i: (0, 0)),
            pl.BlockSpec((1, H), lambda i: (0, 0)),
            pl.BlockSpec((D, H), lambda i: (0, 0)),
            pl.BlockSpec((1, H), lambda i: (0, 0)),
            pl.BlockSpec((H, D), lambda i: (0, 0)),
            pl.BlockSpec((1, D), lambda i: (0, 0)),
            pl.BlockSpec((1, D), lambda i: (0, 0)),
            pl.BlockSpec((1, D), lambda i: (0, 0)),
        ],
        out_specs=pl.BlockSpec((RB, D), lambda i: (i, 0)),
        out_shape=jax.ShapeDtypeStruct((N, D), jnp.float32),
    )(xc, w, g0, g1, W_sum, b_sum.reshape(1, D), W_bi, b_bi.reshape(1, D),
      W_gu, b_gu.reshape(1, H), W_gv, b_gv.reshape(1, H), W_gw,
      b_gw.reshape(1, D), gamma.reshape(1, D), beta.reshape(1, D))


def _split_pad(x):
    xp = jnp.pad(x, ((0, NPAD - N), (0, 0)))
    return jnp.concatenate([xp[:, :DH], xp[:, DH:]], axis=0)


def kernel(X, W_head, b_head, W_tail, b_tail, Wl0, bl0, Wr0, Wl1, bl1, Wr1,
           W_sum, b_sum, W_bi, b_bi, W_gu, b_gu, W_gv, b_gv, W_gw, b_gw,
           gamma, beta):
    e_h, e_t = _heads(X, W_head, b_head, W_tail, b_tail)
    e_tp = jnp.pad(e_t, ((0, NP - N), (0, 0)))
    weights, topk_idx = _topk(e_h, e_tp)

    src = jnp.repeat(jnp.arange(N, dtype=jnp.int32), K)
    dst = topk_idx.reshape(-1)
    edge_index = jnp.stack([src, dst], axis=0)

    idxp = jnp.concatenate(
        [topk_idx, jnp.full((NPAD - N, K), DUMP, jnp.int32)], axis=0)
    idxt = idxp.reshape(NSUB, RT, K).transpose(0, 2, 1).reshape(
        NSUB, K, RT // 128, 128)
    idxf = idxp.reshape(-1).reshape(NSUB * 2, 40, 128)

    zro = jnp.zeros((NPAD, DH), jnp.float32)

    onesf = jnp.ones((NSC * NPAD, DH), jnp.float32)
    cnt = _get_sc_scatter()(onesf, idxt, zro)[:NPAD, 0:1]
    aggr0 = _get_sc_scatter()(_split_pad(X), idxt, zro)
    xc0 = _sage_dense(aggr0[:NPAD], aggr0[NPAD:], cnt, X, Wl0, bl0, Wr0)
    aggr1 = _get_sc_scatter()(_split_pad(xc0), idxt, zro)
    xc1 = _sage_dense(aggr1[:NPAD], aggr1[NPAD:], cnt, xc0, Wl1, bl1, Wr1)

    G = _get_sc_gather()(_split_pad(xc1), idxf)
    g0 = G[:NPAD * K].reshape(NPAD, K, DH)
    g1 = G[NPAD * K:].reshape(NPAD, K, DH)

    out = _final(xc1, weights, g0, g1, W_sum, b_sum, W_bi, b_bi, W_gu, b_gu,
                 W_gv, b_gv, W_gw, b_gw, gamma, beta)
    return (out, edge_index)
